# baseline (speedup 1.0000x reference)
import functools

import jax
import jax.numpy as jnp
from jax import lax
from jax.experimental import pallas as pl
from jax.experimental.pallas import tpu as pltpu
from jax.experimental.pallas import tpu_sc as plsc

EXPERTS = 64
TOPK = 8
TOKENS = 32768
DMODEL = 768
NC, NS = 2, 16
NW = NC * NS
NCHUNK = 4
MM_TILE = 2048


def _logits_body(w_ref, h_ref, out_ref):
    out_ref[...] = lax.dot_general(
        w_ref[...], h_ref[...], (((1,), (1,)), ((), ())),
        preferred_element_type=jnp.float32)


def _logits_t(hf, W, ntok):
    return pl.pallas_call(
        _logits_body,
        grid=(ntok // MM_TILE,),
        in_specs=[
            pl.BlockSpec((EXPERTS, DMODEL), lambda i: (0, 0)),
            pl.BlockSpec((MM_TILE, DMODEL), lambda i: (i, 0)),
        ],
        out_specs=pl.BlockSpec((EXPERTS, MM_TILE), lambda i: (0, i)),
        out_shape=jax.ShapeDtypeStruct((EXPERTS, ntok), jnp.float32),
    )(W, hf)


def _make_topk_tec(tok_per_w):
    groups = tok_per_w // 16

    def _topk_tec(lgT, ids_out, w_out, lg_v, ids_v, w_v):
        c = lax.axis_index("c")
        s = lax.axis_index("s")
        wid = s * NC + c
        base = wid * tok_per_w
        pltpu.sync_copy(lgT.at[:, pl.ds(base, tok_per_w)], lg_v)
        lane8 = lax.iota(jnp.int32, 16) * TOPK

        def group(g, _):
            g16 = g * 16

            def chunk(cidx, carry):
                vs = list(carry[:TOPK])
                ix = list(carry[TOPK:])
                for u in range(8):
                    e = cidx * 8 + u
                    x = lg_v[e, pl.ds(g16, 16)]
                    xi = jnp.full((16,), 1, jnp.int32) * e
                    for j in range(TOPK):
                        cnd = x > vs[j]
                        vs[j], x = jnp.where(cnd, x, vs[j]), jnp.where(cnd, vs[j], x)
                        ix[j], xi = jnp.where(cnd, xi, ix[j]), jnp.where(cnd, ix[j], xi)
                return tuple(vs) + tuple(ix)

            init = tuple(jnp.full((16,), -jnp.inf, jnp.float32) for _ in range(TOPK))
            init = init + tuple(jnp.zeros((16,), jnp.int32) for _ in range(TOPK))
            carry = lax.fori_loop(0, EXPERTS // 8, chunk, init)
            vs = carry[:TOPK]
            ix = carry[TOPK:]
            es = [jnp.exp(v - vs[0]) for v in vs]
            tot = es[0]
            for t in es[1:]:
                tot = tot + t
            pos = lane8 + g * (16 * TOPK)
            for j in range(TOPK):
                plsc.store_scatter(ids_v, [pos + j], ix[j])
                plsc.store_scatter(w_v, [pos + j], es[j] / tot)
            return 0

        lax.fori_loop(0, groups, group, 0)
        pltpu.sync_copy(ids_v, ids_out.at[pl.ds(base * TOPK, tok_per_w * TOPK)])
        pltpu.sync_copy(w_v, w_out.at[pl.ds(base * TOPK, tok_per_w * TOPK)])

    return _topk_tec


def _make_topk_call(ntok):
    tok_per_w = ntok // NW
    return pl.kernel(
        _make_topk_tec(tok_per_w),
        out_type=[
            jax.ShapeDtypeStruct((ntok * TOPK,), jnp.int32),
            jax.ShapeDtypeStruct((ntok * TOPK,), jnp.float32),
        ],
        mesh=plsc.VectorSubcoreMesh(
            core_axis_name="c", subcore_axis_name="s",
            num_cores=NC, num_subcores=NS),
        compiler_params=pltpu.CompilerParams(needs_layout_passes=False),
        scratch_types=[
            pltpu.VMEM((EXPERTS, tok_per_w), jnp.float32),
            pltpu.VMEM((tok_per_w * TOPK,), jnp.int32),
            pltpu.VMEM((tok_per_w * TOPK,), jnp.float32),
        ],
    )


_CHUNK_TOK = TOKENS // NCHUNK
_topk_call = _make_topk_call(_CHUNK_TOK)


def kernel(h, W):
    hf = h.reshape(TOKENS, DMODEL)
    ids_parts = []
    w_parts = []
    for ci in range(NCHUNK):
        hc = lax.slice_in_dim(hf, ci * _CHUNK_TOK, (ci + 1) * _CHUNK_TOK, axis=0)
        lgT = _logits_t(hc, W, _CHUNK_TOK)
        ids_flat, w_flat = _topk_call(lgT)
        ids_parts.append(ids_flat.reshape(_CHUNK_TOK, TOPK))
        w_parts.append(w_flat.reshape(_CHUNK_TOK, TOPK))
    return (jnp.concatenate(ids_parts, axis=0),
            jnp.concatenate(w_parts, axis=0),
            jnp.float32(0.0))


# speedup vs baseline: 1.3643x; 1.3643x over previous
import functools

import jax
import jax.numpy as jnp
from jax import lax
from jax.experimental import pallas as pl
from jax.experimental.pallas import tpu as pltpu
from jax.experimental.pallas import tpu_sc as plsc

EXPERTS = 64
TOPK = 8
TOKENS = 32768
DMODEL = 768
NC, NS = 2, 16
NW = NC * NS
TOK_PER_W = TOKENS // NW
GROUPS = TOK_PER_W // 16
MM_TILE = 2048

_SORT8 = ((0, 1), (2, 3), (4, 5), (6, 7), (0, 2), (1, 3), (4, 6), (5, 7),
          (1, 2), (5, 6), (0, 4), (1, 5), (2, 6), (3, 7), (2, 4), (3, 5),
          (1, 2), (3, 4), (5, 6))
_BITONIC8 = ((0, 4), (1, 5), (2, 6), (3, 7), (0, 2), (1, 3), (4, 6), (5, 7),
             (0, 1), (2, 3), (4, 5), (6, 7))


def _logits_body(w_ref, h_ref, out_ref):
    out_ref[...] = lax.dot_general(
        w_ref[...], h_ref[...], (((1,), (1,)), ((), ())),
        preferred_element_type=jnp.float32)


def _logits_t(hf, W):
    return pl.pallas_call(
        _logits_body,
        grid=(TOKENS // MM_TILE,),
        in_specs=[
            pl.BlockSpec((EXPERTS, DMODEL), lambda i: (0, 0)),
            pl.BlockSpec((MM_TILE, DMODEL), lambda i: (i, 0)),
        ],
        out_specs=pl.BlockSpec((EXPERTS, MM_TILE), lambda i: (0, i)),
        out_shape=jax.ShapeDtypeStruct((EXPERTS, TOKENS), jnp.float32),
    )(W, hf)


def _cswap(v, ix, a, b):
    cnd = v[b] > v[a]
    v[a], v[b] = jnp.where(cnd, v[b], v[a]), jnp.where(cnd, v[a], v[b])
    ix[a], ix[b] = jnp.where(cnd, ix[b], ix[a]), jnp.where(cnd, ix[a], ix[b])


def _topk_tec(lgT, ids_out, w_out, lg_v, ids_v, w_v):
    c = lax.axis_index("c")
    s = lax.axis_index("s")
    wid = s * NC + c
    base = wid * TOK_PER_W
    pltpu.sync_copy(lgT.at[:, pl.ds(base, TOK_PER_W)], lg_v)
    lane8 = lax.iota(jnp.int32, 16) * TOPK

    def group(g, _):
        g16 = g * 16
        vs = [jnp.full((16,), -jnp.inf, jnp.float32) for _ in range(TOPK)]
        ix = [jnp.zeros((16,), jnp.int32) for _ in range(TOPK)]
        for cidx in range(EXPERTS // 8):
            v = [lg_v[cidx * 8 + u, pl.ds(g16, 16)] for u in range(8)]
            vi = [jnp.full((16,), cidx * 8 + u, jnp.int32) for u in range(8)]
            for a, b in _SORT8:
                _cswap(v, vi, a, b)
            for j in range(TOPK):
                cnd = v[7 - j] > vs[j]
                vs[j] = jnp.where(cnd, v[7 - j], vs[j])
                ix[j] = jnp.where(cnd, vi[7 - j], ix[j])
            for a, b in _BITONIC8:
                _cswap(vs, ix, a, b)
        es = [jnp.exp(t - vs[0]) for t in vs]
        tot = es[0]
        for t in es[1:]:
            tot = tot + t
        pos = lane8 + g * (16 * TOPK)
        for j in range(TOPK):
            plsc.store_scatter(ids_v, [pos + j], ix[j])
            plsc.store_scatter(w_v, [pos + j], es[j] / tot)
        return 0

    lax.fori_loop(0, GROUPS, group, 0)
    pltpu.sync_copy(ids_v, ids_out.at[pl.ds(base * TOPK, TOK_PER_W * TOPK)])
    pltpu.sync_copy(w_v, w_out.at[pl.ds(base * TOPK, TOK_PER_W * TOPK)])


_topk_call = pl.kernel(
    _topk_tec,
    out_type=[
        jax.ShapeDtypeStruct((TOKENS * TOPK,), jnp.int32),
        jax.ShapeDtypeStruct((TOKENS * TOPK,), jnp.float32),
    ],
    mesh=plsc.VectorSubcoreMesh(
        core_axis_name="c", subcore_axis_name="s",
        num_cores=NC, num_subcores=NS),
    compiler_params=pltpu.CompilerParams(needs_layout_passes=False),
    scratch_types=[
        pltpu.VMEM((EXPERTS, TOK_PER_W), jnp.float32),
        pltpu.VMEM((TOK_PER_W * TOPK,), jnp.int32),
        pltpu.VMEM((TOK_PER_W * TOPK,), jnp.float32),
    ],
)


def kernel(h, W):
    hf = h.reshape(TOKENS, DMODEL)
    lgT = _logits_t(hf, W)
    ids_flat, w_flat = _topk_call(lgT)
    return (ids_flat.reshape(TOKENS, TOPK),
            w_flat.reshape(TOKENS, TOPK),
            jnp.float32(0.0))


# speedup vs baseline: 1.3784x; 1.0104x over previous
import functools

import jax
import jax.numpy as jnp
from jax import lax
from jax.experimental import pallas as pl
from jax.experimental.pallas import tpu as pltpu
from jax.experimental.pallas import tpu_sc as plsc

EXPERTS = 64
TOPK = 8
TOKENS = 32768
DMODEL = 768
NC, NS = 2, 16
NW = NC * NS
TOK_PER_W = TOKENS // NW
GROUPS = TOK_PER_W // 16
MM_TILE = 4096

_SORT8 = ((0, 1), (2, 3), (4, 5), (6, 7), (0, 2), (1, 3), (4, 6), (5, 7),
          (1, 2), (5, 6), (0, 4), (1, 5), (2, 6), (3, 7), (2, 4), (3, 5),
          (1, 2), (3, 4), (5, 6))
_BITONIC8 = ((0, 4), (1, 5), (2, 6), (3, 7), (0, 2), (1, 3), (4, 6), (5, 7),
             (0, 1), (2, 3), (4, 5), (6, 7))


def _logits_body(w_ref, h_ref, out_ref):
    out_ref[...] = lax.dot_general(
        w_ref[...], h_ref[...], (((1,), (1,)), ((), ())),
        preferred_element_type=jnp.float32)


def _logits_t(hf, W):
    return pl.pallas_call(
        _logits_body,
        grid=(TOKENS // MM_TILE,),
        in_specs=[
            pl.BlockSpec((EXPERTS, DMODEL), lambda i: (0, 0)),
            pl.BlockSpec((MM_TILE, DMODEL), lambda i: (i, 0)),
        ],
        out_specs=pl.BlockSpec((EXPERTS, MM_TILE), lambda i: (0, i)),
        out_shape=jax.ShapeDtypeStruct((EXPERTS, TOKENS), jnp.float32),
    )(W, hf)


def _cswap(v, ix, a, b):
    cnd = v[b] > v[a]
    v[a], v[b] = jnp.where(cnd, v[b], v[a]), jnp.where(cnd, v[a], v[b])
    ix[a], ix[b] = jnp.where(cnd, ix[b], ix[a]), jnp.where(cnd, ix[a], ix[b])


def _topk_tec(lgT, ids_out, w_out, lg_v, ids_v, w_v):
    c = lax.axis_index("c")
    s = lax.axis_index("s")
    wid = s * NC + c
    base = wid * TOK_PER_W
    pltpu.sync_copy(lgT.at[:, pl.ds(base, TOK_PER_W)], lg_v)
    lane8 = lax.iota(jnp.int32, 16) * TOPK

    def group(g, _):
        g16 = g * 16
        vs = [jnp.full((16,), -jnp.inf, jnp.float32) for _ in range(TOPK)]
        ix = [jnp.zeros((16,), jnp.int32) for _ in range(TOPK)]
        for cidx in range(EXPERTS // 8):
            v = [lg_v[cidx * 8 + u, pl.ds(g16, 16)] for u in range(8)]
            vi = [jnp.full((16,), cidx * 8 + u, jnp.int32) for u in range(8)]
            for a, b in _SORT8:
                _cswap(v, vi, a, b)
            for j in range(TOPK):
                cnd = v[7 - j] > vs[j]
                vs[j] = jnp.where(cnd, v[7 - j], vs[j])
                ix[j] = jnp.where(cnd, vi[7 - j], ix[j])
            for a, b in _BITONIC8:
                _cswap(vs, ix, a, b)
        es = [jnp.exp(t - vs[0]) for t in vs]
        tot = es[0]
        for t in es[1:]:
            tot = tot + t
        pos = lane8 + g * (16 * TOPK)
        for j in range(TOPK):
            plsc.store_scatter(ids_v, [pos + j], ix[j])
            plsc.store_scatter(w_v, [pos + j], es[j] / tot)
        return 0

    lax.fori_loop(0, GROUPS, group, 0)
    pltpu.sync_copy(ids_v, ids_out.at[pl.ds(base * TOPK, TOK_PER_W * TOPK)])
    pltpu.sync_copy(w_v, w_out.at[pl.ds(base * TOPK, TOK_PER_W * TOPK)])


_topk_call = pl.kernel(
    _topk_tec,
    out_type=[
        jax.ShapeDtypeStruct((TOKENS * TOPK,), jnp.int32),
        jax.ShapeDtypeStruct((TOKENS * TOPK,), jnp.float32),
    ],
    mesh=plsc.VectorSubcoreMesh(
        core_axis_name="c", subcore_axis_name="s",
        num_cores=NC, num_subcores=NS),
    compiler_params=pltpu.CompilerParams(needs_layout_passes=False),
    scratch_types=[
        pltpu.VMEM((EXPERTS, TOK_PER_W), jnp.float32),
        pltpu.VMEM((TOK_PER_W * TOPK,), jnp.int32),
        pltpu.VMEM((TOK_PER_W * TOPK,), jnp.float32),
    ],
)


def kernel(h, W):
    hf = h.reshape(TOKENS, DMODEL)
    lgT = _logits_t(hf, W)
    ids_flat, w_flat = _topk_call(lgT)
    return (ids_flat.reshape(TOKENS, TOPK),
            w_flat.reshape(TOKENS, TOPK),
            jnp.float32(0.0))
